# Initial kernel scaffold; baseline (speedup 1.0000x reference)
import jax
import jax.numpy as jnp
from jax.experimental import pallas as pl


def kernel(features, samples_0, samples_1, samples_2, W_self_0, W_neigh_0, W_self_1, W_neigh_1):
    raise NotImplementedError("write your pallas kernel here")


# speedup vs baseline: 11.4903x; 11.4903x over previous
import functools

import jax
import jax.numpy as jnp
from jax import lax
from jax.experimental import pallas as pl
from jax.experimental.pallas import tpu as pltpu
from jax.experimental.pallas import tpu_sc as plsc

D = 128
BATCH = 1024
S1 = 10
S2 = 25
N1 = BATCH * S1
LANES = 16
NV = D // LANES

NC = 2
NS = 16
NW = NC * NS

H0_PW = BATCH // NW
H1_PW = N1 // NW
H1_CHUNK = 80
N1_CHUNKS = H1_PW // H1_CHUNK
SEG_PW = N1 // NW
CH_SEGS = 4
CH_ROWS = CH_SEGS * S2
N_CH = SEG_PW // CH_SEGS
NBUF = 4
N_ROUNDS = N_CH // NBUF

_f32 = jnp.float32


def _seg_mean(rows, base, n, scale, out, out_row):
    def body(r, acc):
        return tuple(acc[j] + rows[base + r, pl.ds(LANES * j, LANES)]
                     for j in range(NV))
    acc = lax.fori_loop(0, n, body,
                        tuple(jnp.zeros((LANES,), _f32) for _ in range(NV)))
    for j in range(NV):
        out[out_row, pl.ds(LANES * j, LANES)] = acc[j] * scale


@functools.partial(
    pl.kernel,
    mesh=plsc.VectorSubcoreMesh(core_axis_name="c", subcore_axis_name="s"),
    out_type=(
        jax.ShapeDtypeStruct((BATCH, D), _f32),
        jax.ShapeDtypeStruct((BATCH, D), _f32),
        jax.ShapeDtypeStruct((N1, D), _f32),
        jax.ShapeDtypeStruct((N1, D), _f32),
    ),
    scratch_types=(
        pltpu.VMEM((H0_PW,), jnp.int32),
        pltpu.VMEM((N1_CHUNKS, H1_CHUNK), jnp.int32),
        pltpu.VMEM((N_CH, CH_ROWS), jnp.int32),
        pltpu.VMEM((H0_PW, D), _f32),
        pltpu.VMEM((H1_CHUNK, D), _f32),
        pltpu.VMEM((H0_PW, D), _f32),
        pltpu.VMEM((NBUF, CH_ROWS, D), _f32),
        pltpu.VMEM((NBUF, CH_SEGS, D), _f32),
        pltpu.SemaphoreType.DMA,
        pltpu.SemaphoreType.DMA,
        pltpu.SemaphoreType.DMA,
        pltpu.SemaphoreType.DMA,
        pltpu.SemaphoreType.DMA,
        pltpu.SemaphoreType.DMA,
        pltpu.SemaphoreType.DMA,
        pltpu.SemaphoreType.DMA,
        pltpu.SemaphoreType.DMA,
    ),
)
def _sc_gather(feat, s0, s1, s2, h0_out, m1_out, h1_out, m2_out,
               idx0, idx1, idx2, rows0, rows1, m1b, rows2, m2st,
               sem, semg0, semg1, semg2, semg3,
               semw0, semw1, semw2, semw3):
    semg = (semg0, semg1, semg2, semg3)
    semw = (semw0, semw1, semw2, semw3)
    wid = lax.axis_index("s") * NC + lax.axis_index("c")

    pltpu.sync_copy(s2.at[pl.ds(wid * N_CH, N_CH)], idx2)
    for b in range(NBUF):
        pltpu.async_copy(feat.at[idx2.at[b]], rows2.at[b], semg[b])

    pltpu.sync_copy(s0.at[wid], idx0)
    pltpu.async_copy(feat.at[idx0], rows0, sem).wait()
    pltpu.sync_copy(rows0, h0_out.at[pl.ds(wid * H0_PW, H0_PW)])

    pltpu.sync_copy(s1.at[pl.ds(wid * N1_CHUNKS, N1_CHUNKS)], idx1)
    for c in range(N1_CHUNKS):
        pltpu.async_copy(feat.at[idx1.at[c]], rows1, sem).wait()
        pltpu.sync_copy(
            rows1, h1_out.at[pl.ds(wid * H1_PW + c * H1_CHUNK, H1_CHUNK)])
        for s in range(H1_CHUNK // S1):
            _seg_mean(rows1, S1 * s, S1, 1.0 / S1, m1b, c * (H1_CHUNK // S1) + s)
    pltpu.sync_copy(m1b, m1_out.at[pl.ds(wid * H0_PW, H0_PW)])

    def _m2_write(g, b):
        return pltpu.make_async_copy(
            m2st.at[b],
            m2_out.at[pl.ds(wid * SEG_PW + CH_SEGS * g, CH_SEGS)], semw[b])

    def m2_round(i, carry):
        for b in range(NBUF):
            g = NBUF * i + b
            pltpu.make_async_copy(
                feat.at[idx2.at[g]], rows2.at[b], semg[b]).wait()
            buf = rows2.at[b]

            @pl.when(i > 0)
            def _():
                _m2_write(g - NBUF, b).wait()

            for s in range(CH_SEGS):
                _seg_mean(buf, S2 * s, S2, 1.0 / S2, m2st.at[b], s)
            _m2_write(g, b).start()

            @pl.when(i < N_ROUNDS - 1)
            def _():
                pltpu.async_copy(
                    feat.at[idx2.at[g + NBUF]], rows2.at[b], semg[b])
        return carry

    lax.fori_loop(0, N_ROUNDS, m2_round, 0)
    for b in range(NBUF):
        _m2_write(NBUF * (N_ROUNDS - 1) + b, b).wait()


def _tc_dense(h0, m1, h1, m2, ws0, wn0, ws1, wn1, p0, out):
    dot = lambda a, b: lax.dot(a, b, preferred_element_type=_f32)
    relu = lambda x: jnp.maximum(x, 0.0)
    ws0v = ws0[:]
    wn0v = wn0[:]
    a = relu(dot(h0[:], ws0v))
    b = relu(dot(m1[:], wn0v))
    out_self = dot(a, ws1[0:D, :]) + dot(b, ws1[D:2 * D, :])
    p0v = p0[:]
    ms_blocks, mn_blocks = [], []
    rows_per_blk = 1280
    for t in range(N1 // rows_per_blk):
        sl = pl.ds(t * rows_per_blk, rows_per_blk)
        xs = relu(dot(h1[sl, :], ws0v))
        xn = relu(dot(m2[sl, :], wn0v))
        ms_blocks.append(dot(p0v, xs))
        mn_blocks.append(dot(p0v, xn))
    ms = jnp.concatenate(ms_blocks, axis=0)
    mn = jnp.concatenate(mn_blocks, axis=0)
    out_neigh = dot(ms, wn1[0:D, :]) + dot(mn, wn1[D:2 * D, :])
    out[:] = jnp.concatenate([out_self, out_neigh], axis=1)


_tc_call = pl.pallas_call(
    _tc_dense,
    out_shape=jax.ShapeDtypeStruct((BATCH, 2 * D), _f32),
)


def kernel(features, samples_0, samples_1, samples_2,
           W_self_0, W_neigh_0, W_self_1, W_neigh_1):
    s0 = samples_0.reshape(NW, H0_PW)
    s1 = samples_1.reshape(NW * N1_CHUNKS, H1_CHUNK)
    s2 = samples_2.reshape(NW * N_CH, CH_ROWS)
    h0, m1, h1, m2 = _sc_gather(features, s0, s1, s2)
    blk = 1280
    p0 = (jnp.arange(blk, dtype=jnp.int32)[None, :] // S1
          == jnp.arange(blk // S1, dtype=jnp.int32)[:, None]).astype(_f32) / S1
    return _tc_call(h0, m1, h1, m2, W_self_0, W_neigh_0, W_self_1, W_neigh_1, p0)


# speedup vs baseline: 11.5097x; 1.0017x over previous
import functools

import numpy as np

import jax
import jax.numpy as jnp
from jax import lax
from jax.experimental import pallas as pl
from jax.experimental.pallas import tpu as pltpu
from jax.experimental.pallas import tpu_sc as plsc

D = 128
BATCH = 1024
S1 = 10
S2 = 25
N1 = BATCH * S1
LANES = 16
NV = D // LANES

NC = 2
NS = 16
NW = NC * NS

H0_PW = BATCH // NW
H1_PW = N1 // NW
H1_CHUNK = 80
N1_CHUNKS = H1_PW // H1_CHUNK
SEG_PW = N1 // NW
CH_SEGS = 4
CH_ROWS = CH_SEGS * S2
N_CH = SEG_PW // CH_SEGS
NBUF = 4
N_ROUNDS = N_CH // NBUF

_f32 = jnp.float32


def _seg_mean(rows, base, n, scale, out, out_row):
    def body(r, acc):
        return tuple(acc[j] + rows[base + r, pl.ds(LANES * j, LANES)]
                     for j in range(NV))
    acc = lax.fori_loop(0, n, body,
                        tuple(jnp.zeros((LANES,), _f32) for _ in range(NV)))
    for j in range(NV):
        out[out_row, pl.ds(LANES * j, LANES)] = acc[j] * scale


@functools.partial(
    pl.kernel,
    mesh=plsc.VectorSubcoreMesh(core_axis_name="c", subcore_axis_name="s"),
    out_type=(
        jax.ShapeDtypeStruct((BATCH, D), _f32),
        jax.ShapeDtypeStruct((BATCH, D), _f32),
        jax.ShapeDtypeStruct((N1, D), _f32),
        jax.ShapeDtypeStruct((N1, D), _f32),
    ),
    scratch_types=(
        pltpu.VMEM((H0_PW,), jnp.int32),
        pltpu.VMEM((N1_CHUNKS, H1_CHUNK), jnp.int32),
        pltpu.VMEM((N_CH, CH_ROWS), jnp.int32),
        pltpu.VMEM((H0_PW, D), _f32),
        pltpu.VMEM((H1_CHUNK, D), _f32),
        pltpu.VMEM((H0_PW, D), _f32),
        pltpu.VMEM((NBUF, CH_ROWS, D), _f32),
        pltpu.VMEM((NBUF, CH_SEGS, D), _f32),
        pltpu.SemaphoreType.DMA,
        pltpu.SemaphoreType.DMA,
        pltpu.SemaphoreType.DMA,
        pltpu.SemaphoreType.DMA,
        pltpu.SemaphoreType.DMA,
        pltpu.SemaphoreType.DMA,
        pltpu.SemaphoreType.DMA,
        pltpu.SemaphoreType.DMA,
        pltpu.SemaphoreType.DMA,
    ),
)
def _sc_gather(feat, s0, s1, s2, h0_out, m1_out, h1_out, m2_out,
               idx0, idx1, idx2, rows0, rows1, m1b, rows2, m2st,
               sem, semg0, semg1, semg2, semg3,
               semw0, semw1, semw2, semw3):
    semg = (semg0, semg1, semg2, semg3)
    semw = (semw0, semw1, semw2, semw3)
    wid = lax.axis_index("s") * NC + lax.axis_index("c")

    pltpu.sync_copy(s2.at[pl.ds(wid * N_CH, N_CH)], idx2)
    for b in range(NBUF):
        pltpu.async_copy(feat.at[idx2.at[b]], rows2.at[b], semg[b])

    pltpu.sync_copy(s0.at[wid], idx0)
    pltpu.async_copy(feat.at[idx0], rows0, sem).wait()
    pltpu.sync_copy(rows0, h0_out.at[pl.ds(wid * H0_PW, H0_PW)])

    pltpu.sync_copy(s1.at[pl.ds(wid * N1_CHUNKS, N1_CHUNKS)], idx1)
    for c in range(N1_CHUNKS):
        pltpu.async_copy(feat.at[idx1.at[c]], rows1, sem).wait()
        pltpu.sync_copy(
            rows1, h1_out.at[pl.ds(wid * H1_PW + c * H1_CHUNK, H1_CHUNK)])
        for s in range(H1_CHUNK // S1):
            _seg_mean(rows1, S1 * s, S1, 1.0 / S1, m1b, c * (H1_CHUNK // S1) + s)
    pltpu.sync_copy(m1b, m1_out.at[pl.ds(wid * H0_PW, H0_PW)])

    def _m2_write(g, b):
        return pltpu.make_async_copy(
            m2st.at[b],
            m2_out.at[pl.ds(wid * SEG_PW + CH_SEGS * g, CH_SEGS)], semw[b])

    def m2_round(i, carry):
        for b in range(NBUF):
            g = NBUF * i + b
            pltpu.make_async_copy(
                feat.at[idx2.at[g]], rows2.at[b], semg[b]).wait()
            buf = rows2.at[b]

            @pl.when(i > 0)
            def _():
                _m2_write(g - NBUF, b).wait()

            for s in range(CH_SEGS):
                _seg_mean(buf, S2 * s, S2, 1.0 / S2, m2st.at[b], s)
            _m2_write(g, b).start()

            @pl.when(i < N_ROUNDS - 1)
            def _():
                pltpu.async_copy(
                    feat.at[idx2.at[g + NBUF]], rows2.at[b], semg[b])
        return carry

    lax.fori_loop(0, N_ROUNDS, m2_round, 0)
    for b in range(NBUF):
        _m2_write(NBUF * (N_ROUNDS - 1) + b, b).wait()


def _tc_dense(h0, m1, h1, m2, ws0, wn0, ws1, wn1, p0, out):
    dot = lambda a, b: lax.dot(a, b, preferred_element_type=_f32)
    relu = lambda x: jnp.maximum(x, 0.0)
    ws0v = ws0[:]
    wn0v = wn0[:]
    a = relu(dot(h0[:], ws0v))
    b = relu(dot(m1[:], wn0v))
    out_self = dot(a, ws1[0:D, :]) + dot(b, ws1[D:2 * D, :])
    p0v = p0[:]
    ms_blocks, mn_blocks = [], []
    rows_per_blk = 1280
    for t in range(N1 // rows_per_blk):
        sl = pl.ds(t * rows_per_blk, rows_per_blk)
        xs = relu(dot(h1[sl, :], ws0v))
        xn = relu(dot(m2[sl, :], wn0v))
        ms_blocks.append(dot(p0v, xs))
        mn_blocks.append(dot(p0v, xn))
    ms = jnp.concatenate(ms_blocks, axis=0)
    mn = jnp.concatenate(mn_blocks, axis=0)
    out_neigh = dot(ms, wn1[0:D, :]) + dot(mn, wn1[D:2 * D, :])
    out[:] = jnp.concatenate([out_self, out_neigh], axis=1)


_tc_call = pl.pallas_call(
    _tc_dense,
    out_shape=jax.ShapeDtypeStruct((BATCH, 2 * D), _f32),
)

_P0 = jnp.asarray(
    (np.arange(1280)[None, :] // S1 == np.arange(128)[:, None]) / S1,
    dtype=_f32)


def kernel(features, samples_0, samples_1, samples_2,
           W_self_0, W_neigh_0, W_self_1, W_neigh_1):
    s0 = samples_0.reshape(NW, H0_PW)
    s1 = samples_1.reshape(NW * N1_CHUNKS, H1_CHUNK)
    s2 = samples_2.reshape(NW * N_CH, CH_ROWS)
    h0, m1, h1, m2 = _sc_gather(features, s0, s1, s2)
    return _tc_call(h0, m1, h1, m2, W_self_0, W_neigh_0, W_self_1, W_neigh_1,
                    _P0)


# speedup vs baseline: 11.6047x; 1.0083x over previous
import functools

import numpy as np

import jax
import jax.numpy as jnp
from jax import lax
from jax.experimental import pallas as pl
from jax.experimental.pallas import tpu as pltpu
from jax.experimental.pallas import tpu_sc as plsc

D = 128
BATCH = 1024
S1 = 10
S2 = 25
N1 = BATCH * S1
LANES = 16
NV = D // LANES

NC = 2
NS = 16
NW = NC * NS

H0_PW = BATCH // NW
H1_PW = N1 // NW
H1_CHUNK = 80
N1_CHUNKS = H1_PW // H1_CHUNK
SEG_PW = N1 // NW
CH_SEGS = 8
CH_ROWS = CH_SEGS * S2
CH_SPLIT = 104
N_CH = SEG_PW // CH_SEGS
NBUF = 2
N_ROUNDS = N_CH // NBUF

_f32 = jnp.float32


def _seg_mean(rows, base, n, scale, out, out_row):
    def body(r, acc):
        return tuple(acc[j] + rows[base + r, pl.ds(LANES * j, LANES)]
                     for j in range(NV))
    acc = lax.fori_loop(0, n, body,
                        tuple(jnp.zeros((LANES,), _f32) for _ in range(NV)))
    for j in range(NV):
        out[out_row, pl.ds(LANES * j, LANES)] = acc[j] * scale


@functools.partial(
    pl.kernel,
    mesh=plsc.VectorSubcoreMesh(core_axis_name="c", subcore_axis_name="s"),
    out_type=(
        jax.ShapeDtypeStruct((BATCH, D), _f32),
        jax.ShapeDtypeStruct((BATCH, D), _f32),
        jax.ShapeDtypeStruct((N1, D), _f32),
        jax.ShapeDtypeStruct((N1, D), _f32),
    ),
    scratch_types=(
        pltpu.VMEM((H0_PW,), jnp.int32),
        pltpu.VMEM((H1_PW,), jnp.int32),
        pltpu.VMEM((SEG_PW * S2,), jnp.int32),
        pltpu.VMEM((H0_PW, D), _f32),
        pltpu.VMEM((H1_CHUNK, D), _f32),
        pltpu.VMEM((H0_PW, D), _f32),
        pltpu.VMEM((NBUF, CH_ROWS, D), _f32),
        pltpu.VMEM((NBUF, CH_SEGS, D), _f32),
        pltpu.SemaphoreType.DMA,
        pltpu.SemaphoreType.DMA,
        pltpu.SemaphoreType.DMA,
        pltpu.SemaphoreType.DMA,
        pltpu.SemaphoreType.DMA,
    ),
)
def _sc_gather(feat, s0, s1, s2, h0_out, m1_out, h1_out, m2_out,
               idx0, idx1, idx2, rows0, rows1, m1b, rows2, m2st,
               sem, semg0, semg1, semw0, semw1):
    semg = (semg0, semg1)
    semw = (semw0, semw1)
    wid = lax.axis_index("s") * NC + lax.axis_index("c")

    def _m2_gathers(g, b):
        base = g * CH_ROWS
        lo = pltpu.make_async_copy(
            feat.at[idx2.at[pl.ds(base, CH_SPLIT)]],
            rows2.at[b].at[pl.ds(0, CH_SPLIT)], semg[b])
        hi = pltpu.make_async_copy(
            feat.at[idx2.at[pl.ds(base + CH_SPLIT, CH_ROWS - CH_SPLIT)]],
            rows2.at[b].at[pl.ds(CH_SPLIT, CH_ROWS - CH_SPLIT)], semg[b])
        return lo, hi

    pltpu.sync_copy(s2.at[pl.ds(wid * (SEG_PW * S2), SEG_PW * S2)], idx2)
    for b in range(NBUF):
        for cp in _m2_gathers(b, b):
            cp.start()

    pltpu.sync_copy(s0.at[pl.ds(wid * H0_PW, H0_PW)], idx0)
    pltpu.async_copy(feat.at[idx0], rows0, sem).wait()
    pltpu.sync_copy(rows0, h0_out.at[pl.ds(wid * H0_PW, H0_PW)])

    pltpu.sync_copy(s1.at[pl.ds(wid * H1_PW, H1_PW)], idx1)
    for c in range(N1_CHUNKS):
        pltpu.async_copy(
            feat.at[idx1.at[pl.ds(c * H1_CHUNK, H1_CHUNK)]], rows1,
            sem).wait()
        pltpu.sync_copy(
            rows1, h1_out.at[pl.ds(wid * H1_PW + c * H1_CHUNK, H1_CHUNK)])

        def m1_seg(s, carry):
            _seg_mean(rows1, S1 * s, S1, 1.0 / S1, m1b,
                      c * (H1_CHUNK // S1) + s)
            return carry

        lax.fori_loop(0, H1_CHUNK // S1, m1_seg, 0)
    pltpu.sync_copy(m1b, m1_out.at[pl.ds(wid * H0_PW, H0_PW)])

    def _m2_write(g, b):
        return pltpu.make_async_copy(
            m2st.at[b],
            m2_out.at[pl.ds(wid * SEG_PW + CH_SEGS * g, CH_SEGS)], semw[b])

    def m2_round(i, carry):
        for b in range(NBUF):
            g = NBUF * i + b
            for cp in _m2_gathers(g, b):
                cp.wait()
            buf = rows2.at[b]

            @pl.when(i > 0)
            def _():
                _m2_write(g - NBUF, b).wait()

            def m2_seg(s, carry2):
                _seg_mean(buf, S2 * s, S2, 1.0 / S2, m2st.at[b], s)
                return carry2

            lax.fori_loop(0, CH_SEGS, m2_seg, 0)
            _m2_write(g, b).start()

            @pl.when(i < N_ROUNDS - 1)
            def _():
                for cp in _m2_gathers(g + NBUF, b):
                    cp.start()
        return carry

    lax.fori_loop(0, N_ROUNDS, m2_round, 0)
    for b in range(NBUF):
        _m2_write(NBUF * (N_ROUNDS - 1) + b, b).wait()


def _tc_dense(h0, m1, h1, m2, ws0, wn0, ws1, wn1, p0, out):
    dot = lambda a, b: lax.dot(a, b, preferred_element_type=_f32)
    relu = lambda x: jnp.maximum(x, 0.0)
    ws0v = ws0[:]
    wn0v = wn0[:]
    a = relu(dot(h0[:], ws0v))
    b = relu(dot(m1[:], wn0v))
    out_self = dot(a, ws1[0:D, :]) + dot(b, ws1[D:2 * D, :])
    p0v = p0[:]
    ms_blocks, mn_blocks = [], []
    rows_per_blk = 1280
    for t in range(N1 // rows_per_blk):
        sl = pl.ds(t * rows_per_blk, rows_per_blk)
        xs = relu(dot(h1[sl, :], ws0v))
        xn = relu(dot(m2[sl, :], wn0v))
        ms_blocks.append(dot(p0v, xs))
        mn_blocks.append(dot(p0v, xn))
    ms = jnp.concatenate(ms_blocks, axis=0)
    mn = jnp.concatenate(mn_blocks, axis=0)
    out_neigh = dot(ms, wn1[0:D, :]) + dot(mn, wn1[D:2 * D, :])
    out[:] = jnp.concatenate([out_self, out_neigh], axis=1)


_tc_call = pl.pallas_call(
    _tc_dense,
    out_shape=jax.ShapeDtypeStruct((BATCH, 2 * D), _f32),
)

_P0 = np.asarray(
    (np.arange(1280)[None, :] // S1 == np.arange(128)[:, None]) / S1,
    dtype=np.float32)


def kernel(features, samples_0, samples_1, samples_2,
           W_self_0, W_neigh_0, W_self_1, W_neigh_1):
    h0, m1, h1, m2 = _sc_gather(features, samples_0, samples_1, samples_2)
    return _tc_call(h0, m1, h1, m2, W_self_0, W_neigh_0, W_self_1, W_neigh_1,
                    _P0)
